# baseline (speedup 1.0000x reference)
import jax
import jax.numpy as jnp
from jax import lax
from jax.experimental import pallas as pl
from jax.experimental.pallas import tpu as pltpu

_C = 1000
_P = 16
_D = 512
_ROWS = _C * _P


def _tc_body(in_ref, out_ref, idx_ref, sem):
    cp = pltpu.make_async_copy(in_ref, out_ref, sem)
    cp.start()
    i = lax.broadcasted_iota(jnp.int32, (125, 128), 0)
    j = lax.broadcasted_iota(jnp.int32, (125, 128), 1)
    idx_ref[...] = (i * 128 + j) >> 4
    cp.wait()


def kernel(vision_protos):
    stacked, idx2d = pl.pallas_call(
        _tc_body,
        in_specs=[pl.BlockSpec(memory_space=pl.ANY)],
        out_specs=[pl.BlockSpec(memory_space=pl.ANY),
                   pl.BlockSpec((125, 128), lambda: (0, 0))],
        out_shape=[jax.ShapeDtypeStruct((_C, _P, _D), jnp.float32),
                   jax.ShapeDtypeStruct((125, 128), jnp.int32)],
        scratch_shapes=[pltpu.SemaphoreType.DMA],
    )(vision_protos)
    return (stacked.reshape(_ROWS, _D), idx2d.reshape(_ROWS))


# speedup vs baseline: 11.1953x; 11.1953x over previous
import jax
import jax.numpy as jnp
from jax import lax
from jax.experimental import pallas as pl
from jax.experimental.pallas import tpu as pltpu
from jax.experimental.pallas import tpu_sc as plsc

_C = 1000
_P = 16
_D = 512
_ROWS = _C * _P
_NC = 2
_NS = 16
_NW = _NC * _NS

_TC_CLS = 660
_SC_CLS = _C - _TC_CLS
_CHUNK = 4
_SC_NCHUNKS = _SC_CLS // _CHUNK
_JMAX = -(-_SC_NCHUNKS // _NW)

assert _SC_CLS % _CHUNK == 0

_TC_BLK = 60
assert _TC_CLS % _TC_BLK == 0


def _sc_body(protos_hbm, out_hbm, idx_hbm, buf, idx_v, r0, r1, w0, w1):
    wid = lax.axis_index("s") * _NC + lax.axis_index("c")
    rsems = (r0, r1)
    wsems = (w0, w1)

    def rd(j, b):
        c0 = _TC_CLS + _CHUNK * (wid + _NW * j)
        return pltpu.make_async_copy(protos_hbm.at[pl.ds(c0, _CHUNK)],
                                     buf.at[b], rsems[b])

    def wr(j, b):
        c0 = _TC_CLS + _CHUNK * (wid + _NW * j)
        return pltpu.make_async_copy(buf.at[b],
                                     out_hbm.at[pl.ds(c0, _CHUNK)], wsems[b])

    def chunk_ok(j):
        return wid < (_SC_NCHUNKS - _NW * j)

    def guarded(j, mk):
        if _NW * (j + 1) <= _SC_NCHUNKS:
            mk()
        else:
            @pl.when(chunk_ok(j))
            def _():
                mk()

    guarded(0, lambda: rd(0, 0).start())

    for i in range(32):
        idx_v[pl.ds(_P * i, _P)] = jnp.full((_P,), 32 * wid + i, jnp.int32)

    @pl.when(wid < _NW - 1)
    def _():
        pltpu.sync_copy(idx_v, idx_hbm.at[pl.ds(512 * wid, 512)])

    @pl.when(wid == _NW - 1)
    def _():
        pltpu.sync_copy(idx_v.at[pl.ds(0, 128)],
                        idx_hbm.at[pl.ds(512 * (_NW - 1), 128)])

    for j in range(_JMAX):
        b = j % 2
        guarded(j, lambda: rd(j, b).wait())
        if j >= 1:
            guarded(j - 1, lambda: wr(j - 1, 1 - b).wait())
        if j + 1 < _JMAX:
            guarded(j + 1, lambda: rd(j + 1, 1 - b).start())
        guarded(j, lambda: wr(j, b).start())
    guarded(_JMAX - 1, lambda: wr(_JMAX - 1, (_JMAX - 1) % 2).wait())


def _tc_body(in_ref, out_ref):
    out_ref[...] = in_ref[...]


def kernel(vision_protos):
    sc = pl.kernel(
        _sc_body,
        out_type=(jax.ShapeDtypeStruct((_C, _P, _D), jnp.float32),
                  jax.ShapeDtypeStruct((_ROWS,), jnp.int32)),
        mesh=plsc.VectorSubcoreMesh(core_axis_name="c", subcore_axis_name="s"),
        scratch_types=[
            pltpu.VMEM((2, _CHUNK, _P, _D), jnp.float32),
            pltpu.VMEM((512,), jnp.int32),
            pltpu.SemaphoreType.DMA,
            pltpu.SemaphoreType.DMA,
            pltpu.SemaphoreType.DMA,
            pltpu.SemaphoreType.DMA,
        ],
    )
    tail, class_idx = sc(vision_protos)

    head = pl.pallas_call(
        _tc_body,
        grid=(_TC_CLS // _TC_BLK,),
        in_specs=[pl.BlockSpec((_TC_BLK, _P, _D), lambda g: (g, 0, 0))],
        out_specs=pl.BlockSpec((_TC_BLK, _P, _D), lambda g: (g, 0, 0)),
        out_shape=jax.ShapeDtypeStruct((_TC_CLS, _P, _D), jnp.float32),
    )(vision_protos[:_TC_CLS])

    stacked = lax.concatenate(
        [head.reshape(_TC_CLS * _P, _D),
         tail.reshape(_C * _P, _D)[_TC_CLS * _P:]], 0)
    return (stacked, class_idx)


# speedup vs baseline: 26.0037x; 2.3227x over previous
import jax
import jax.numpy as jnp
from jax import lax
from jax.experimental import pallas as pl
from jax.experimental.pallas import tpu as pltpu
from jax.experimental.pallas import tpu_sc as plsc

_C = 1000
_P = 16
_D = 512
_ROWS = _C * _P
_NC = 2
_NS = 16
_NW = _NC * _NS

_TC_BLK = 125


def _sc_idx_body(idx_hbm, idx_v):
    wid = lax.axis_index("s") * _NC + lax.axis_index("c")
    for i in range(32):
        idx_v[pl.ds(_P * i, _P)] = jnp.full((_P,), 32 * wid + i, jnp.int32)

    @pl.when(wid < _NW - 1)
    def _():
        pltpu.sync_copy(idx_v, idx_hbm.at[pl.ds(512 * wid, 512)])

    @pl.when(wid == _NW - 1)
    def _():
        pltpu.sync_copy(idx_v.at[pl.ds(0, 128)],
                        idx_hbm.at[pl.ds(512 * (_NW - 1), 128)])


def _tc_copy_body(in_ref, out_ref):
    out_ref[...] = in_ref[...]


def kernel(vision_protos):
    class_idx = pl.kernel(
        _sc_idx_body,
        out_type=jax.ShapeDtypeStruct((_ROWS,), jnp.int32),
        mesh=plsc.VectorSubcoreMesh(core_axis_name="c", subcore_axis_name="s"),
        scratch_types=[pltpu.VMEM((512,), jnp.int32)],
    )()

    stacked = pl.pallas_call(
        _tc_copy_body,
        grid=(_C // _TC_BLK,),
        in_specs=[pl.BlockSpec((_TC_BLK, _P, _D), lambda g: (g, 0, 0))],
        out_specs=pl.BlockSpec((_TC_BLK, _P, _D), lambda g: (g, 0, 0)),
        out_shape=jax.ShapeDtypeStruct((_C, _P, _D), jnp.float32),
    )(vision_protos)

    return (stacked.reshape(_ROWS, _D), class_idx)
